# Initial kernel scaffold; baseline (speedup 1.0000x reference)
import jax
import jax.numpy as jnp
from jax.experimental import pallas as pl


def kernel(x, edge_index, batch, W1, b1, W2, b2, W3, b3, Wg1, bg1, Wg2, bg2):
    raise NotImplementedError("write your pallas kernel here")


# speedup vs baseline: 6.1560x; 6.1560x over previous
import functools

import jax
import jax.numpy as jnp
from jax import lax
from jax.experimental import pallas as pl
from jax.experimental.pallas import tpu as pltpu
from jax.experimental.pallas import tpu_sc as plsc

NNODE = 10000
NPAD = 10240
ROWB = 2048
NRB = NPAD // ROWB
NSEG = 256
EDGES = 320000
E2 = EDGES + NNODE
EPW = 10368
E2P = EPW * 32
STEPS = EPW // 128
ROWS_PER_SUB = NPAD // 16


def _sc_scatter_body(nch, *refs):
    g_refs = refs[:nch]
    src_hbm, dst_hbm, zeros_hbm = refs[nch:nch + 3]
    out_refs = refs[nch + 3:2 * nch + 3]
    src_v, dst_v, rows_v, sem, acc = refs[2 * nch + 3:]

    c = lax.axis_index("c")
    sid = lax.axis_index("s")
    ebase = pl.multiple_of(c * (16 * EPW) + sid * EPW, 128)
    rbase = pl.multiple_of(sid * ROWS_PER_SUB, 8)

    for ch in range(nch):
        pltpu.sync_copy(zeros_hbm.at[pl.ds(rbase, ROWS_PER_SUB)],
                        acc.at[pl.ds(rbase, ROWS_PER_SUB)])
        plsc.subcore_barrier()

        def step(s, _):
            off = pl.multiple_of(ebase + s * 128, 128)
            pltpu.sync_copy(src_hbm.at[pl.ds(off, 128)], src_v)
            pltpu.sync_copy(dst_hbm.at[pl.ds(off, 128)], dst_v)
            pltpu.async_copy(g_refs[ch].at[src_v], rows_v, sem).wait()
            pltpu.sync_copy(rows_v, acc.at[dst_v], add=True)
            return 0

        lax.fori_loop(0, STEPS, step, 0)
        plsc.subcore_barrier()
        pltpu.sync_copy(acc.at[pl.ds(rbase, ROWS_PER_SUB)],
                        out_refs[ch].at[c, pl.ds(rbase, ROWS_PER_SUB)])
        plsc.subcore_barrier()


def _sc_scatter(nch, g_list, src, dst, zeros):
    mesh = plsc.VectorSubcoreMesh(core_axis_name="c", subcore_axis_name="s")
    kern = pl.kernel(
        functools.partial(_sc_scatter_body, nch),
        out_type=[jax.ShapeDtypeStruct((2, NPAD, 128), jnp.float32)
                  for _ in range(nch)],
        mesh=mesh,
        scratch_types=[
            pltpu.VMEM((128,), jnp.int32),
            pltpu.VMEM((128,), jnp.int32),
            pltpu.VMEM((128, 128), jnp.float32),
            pltpu.SemaphoreType.DMA,
            pltpu.VMEM_SHARED((NPAD, 128), jnp.float32),
        ],
    )
    return kern(*g_list, src, dst, zeros)


def _sc_deg_body(dst_hbm, ones_hbm, zeros_hbm, out_ref,
                 dst_v, ones_v, acc):
    c = lax.axis_index("c")
    sid = lax.axis_index("s")
    ebase = pl.multiple_of(c * (16 * EPW) + sid * EPW, 128)
    rbase = pl.multiple_of(sid * ROWS_PER_SUB, 8)

    pltpu.sync_copy(ones_hbm, ones_v)
    pltpu.sync_copy(zeros_hbm.at[pl.ds(rbase, ROWS_PER_SUB)],
                    acc.at[pl.ds(rbase, ROWS_PER_SUB)])
    plsc.subcore_barrier()

    def step(s, _):
        off = pl.multiple_of(ebase + s * 128, 128)
        pltpu.sync_copy(dst_hbm.at[pl.ds(off, 128)], dst_v)
        pltpu.sync_copy(ones_v, acc.at[dst_v], add=True)
        return 0

    lax.fori_loop(0, STEPS, step, 0)
    plsc.subcore_barrier()
    pltpu.sync_copy(acc.at[pl.ds(rbase, ROWS_PER_SUB)],
                    out_ref.at[c, pl.ds(rbase, ROWS_PER_SUB)])


def _sc_deg(dst, ones, zeros):
    mesh = plsc.VectorSubcoreMesh(core_axis_name="c", subcore_axis_name="s")
    kern = pl.kernel(
        _sc_deg_body,
        out_type=jax.ShapeDtypeStruct((2, NPAD, 128), jnp.float32),
        mesh=mesh,
        scratch_types=[
            pltpu.VMEM((128,), jnp.int32),
            pltpu.VMEM((128, 128), jnp.float32),
            pltpu.VMEM_SHARED((NPAD, 128), jnp.float32),
        ],
    )
    return kern(dst, ones, zeros)


def _dinv_body(deg_ref, dinv_ref):
    d = deg_ref[0] + deg_ref[1]
    dinv_ref[...] = jnp.where(d > 0, lax.rsqrt(d), 0.0)


def _dinv(deg):
    return pl.pallas_call(
        _dinv_body,
        grid=(NRB,),
        in_specs=[pl.BlockSpec((2, ROWB, 128), lambda r: (0, r, 0))],
        out_specs=pl.BlockSpec((ROWB, 128), lambda r: (r, 0)),
        out_shape=jax.ShapeDtypeStruct((NPAD, 128), jnp.float32),
    )(deg)


def _mm_body(nch, h_ref, w_ref, dinv_ref, g_ref):
    acc = jnp.dot(h_ref[...], w_ref[...],
                  preferred_element_type=jnp.float32,
                  precision=lax.Precision.HIGHEST)
    dinv = dinv_ref[...]
    for ch in range(nch):
        g_ref[ch] = acc[:, ch * 128:(ch + 1) * 128] * dinv


def _mm_scale(h, w, dinv):
    k, dout = w.shape
    nch = dout // 128
    return pl.pallas_call(
        functools.partial(_mm_body, nch),
        grid=(NRB,),
        in_specs=[
            pl.BlockSpec((ROWB, k), lambda r: (r, 0)),
            pl.BlockSpec((k, dout), lambda r: (0, 0)),
            pl.BlockSpec((ROWB, 128), lambda r: (r, 0)),
        ],
        out_specs=pl.BlockSpec((nch, ROWB, 128), lambda r: (0, r, 0)),
        out_shape=jax.ShapeDtypeStruct((nch, NPAD, 128), jnp.float32),
    )(h, w, dinv)


def _epi_body(nch, *refs):
    p_refs = refs[:nch]
    b_ref, dinv_ref = refs[nch], refs[nch + 1]
    out_ref = refs[nch + 2]
    dinv = dinv_ref[...]
    for ch in range(nch):
        s = p_refs[ch][0] + p_refs[ch][1]
        out_ref[:, ch * 128:(ch + 1) * 128] = jax.nn.relu(
            dinv * s + b_ref[0, ch * 128:(ch + 1) * 128][None, :])


def _epilogue(p_list, b, dinv):
    nch = len(p_list)
    dout = nch * 128
    return pl.pallas_call(
        functools.partial(_epi_body, nch),
        grid=(NRB,),
        in_specs=(
            [pl.BlockSpec((2, ROWB, 128), lambda r: (0, r, 0))
             for _ in range(nch)]
            + [pl.BlockSpec((1, dout), lambda r: (0, 0)),
               pl.BlockSpec((ROWB, 128), lambda r: (r, 0))]
        ),
        out_specs=pl.BlockSpec((ROWB, dout), lambda r: (r, 0)),
        out_shape=jax.ShapeDtypeStruct((NPAD, dout), jnp.float32),
    )(*p_list, b.reshape(1, dout), dinv)


def _pool_body(h_ref, batch_ref, out_ref):
    r = pl.program_id(0)

    @pl.when(r == 0)
    def _():
        out_ref[...] = jnp.zeros_like(out_ref)

    h = h_ref[...]
    b = batch_ref[...]
    smin = jnp.min(b)
    smax = jnp.minimum(jnp.max(b), NSEG - 1)
    segcol = lax.broadcasted_iota(jnp.int32, (NSEG, 1), 0)

    def body(g, res):
        mrow = jnp.max(jnp.where(b == g, h, 0.0), axis=0, keepdims=True)
        return jnp.where(segcol == g, jnp.maximum(res, mrow), res)

    res = lax.fori_loop(smin, smax + 1, body,
                        jnp.zeros_like(out_ref[...]))
    out_ref[...] = jnp.maximum(out_ref[...], res)


def _pool(h3, batch2d):
    return pl.pallas_call(
        _pool_body,
        grid=(NRB,),
        in_specs=[pl.BlockSpec((ROWB, 512), lambda r: (r, 0)),
                  pl.BlockSpec((ROWB, 1), lambda r: (r, 0))],
        out_specs=pl.BlockSpec((NSEG, 512), lambda r: (0, 0)),
        out_shape=jax.ShapeDtypeStruct((NSEG, 512), jnp.float32),
    )(h3, batch2d)


def _mlp_body(p_ref, w1_ref, b1_ref, w2_ref, b2_ref, out_ref):
    z = jax.nn.relu(
        jnp.dot(p_ref[...], w1_ref[...],
                preferred_element_type=jnp.float32,
                precision=lax.Precision.HIGHEST) + b1_ref[...])
    out_ref[...] = jax.nn.relu(
        jnp.dot(z, w2_ref[...],
                preferred_element_type=jnp.float32,
                precision=lax.Precision.HIGHEST) + b2_ref[...])


def _mlp(pooled, wg1, bg1, wg2, bg2):
    return pl.pallas_call(
        _mlp_body,
        out_shape=jax.ShapeDtypeStruct((NSEG, 128), jnp.float32),
    )(pooled, wg1, bg1.reshape(1, 1024), wg2, bg2.reshape(1, 128))


def kernel(x, edge_index, batch, W1, b1, W2, b2, W3, b3, Wg1, bg1, Wg2, bg2):
    loop = jnp.arange(NNODE, dtype=edge_index.dtype)
    padv = jnp.full((E2P - E2,), NNODE, dtype=edge_index.dtype)
    src = jnp.concatenate([edge_index[0], loop, padv]).astype(jnp.int32)
    dst = jnp.concatenate([edge_index[1], loop, padv]).astype(jnp.int32)

    xp = jnp.zeros((NPAD, 128), x.dtype).at[:NNODE].set(x)
    zeros = jnp.zeros((NPAD, 128), jnp.float32)
    ones = jnp.ones((128, 128), jnp.float32)
    batch2d = jnp.full((NPAD, 1), NSEG, jnp.int32).at[:NNODE, 0].set(
        batch.astype(jnp.int32))

    deg = _sc_deg(dst, ones, zeros)
    dinv = _dinv(deg)

    h = xp
    for w, b in ((W1, b1), (W2, b2), (W3, b3)):
        nch = w.shape[1] // 128
        g = _mm_scale(h, w, dinv)
        g_list = [g[ch] for ch in range(nch)]
        p_list = _sc_scatter(nch, g_list, src, dst, zeros)
        if not isinstance(p_list, (list, tuple)):
            p_list = [p_list]
        p_list = list(p_list)
        h = _epilogue(p_list, b, dinv)

    pooled = _pool(h, batch2d)
    return _mlp(pooled, Wg1, bg1, Wg2, bg2)


# speedup vs baseline: 8.0798x; 1.3125x over previous
import functools

import jax
import jax.numpy as jnp
from jax import lax
from jax.experimental import pallas as pl
from jax.experimental.pallas import tpu as pltpu
from jax.experimental.pallas import tpu_sc as plsc

NNODE = 10000
NPAD = 10240
ROWB = 2048
NRB = NPAD // ROWB
NSEG = 256
EDGES = 320000
E2 = EDGES + NNODE
EPW = 10368
E2P = EPW * 32
STEPS = EPW // 128
ROWS_PER_SUB = NPAD // 16


def _sc_scatter_body(nch, *refs):
    g_refs = refs[:nch]
    src_hbm, dst_hbm, zeros_hbm = refs[nch:nch + 3]
    out_refs = refs[nch + 3:2 * nch + 3]
    src_v = refs[2 * nch + 3:2 * nch + 5]
    dst_v = refs[2 * nch + 5:2 * nch + 7]
    rows_v = refs[2 * nch + 7:2 * nch + 9]
    sems = refs[2 * nch + 9:2 * nch + 11]
    acc = refs[2 * nch + 11]

    c = lax.axis_index("c")
    sid = lax.axis_index("s")
    ebase = pl.multiple_of(c * (16 * EPW) + sid * EPW, 128)
    rbase = pl.multiple_of(sid * ROWS_PER_SUB, 8)

    for ch in range(nch):
        g = g_refs[ch]
        pltpu.sync_copy(zeros_hbm.at[pl.ds(rbase, ROWS_PER_SUB)],
                        acc.at[pl.ds(rbase, ROWS_PER_SUB)])
        plsc.subcore_barrier()

        pltpu.sync_copy(src_hbm.at[pl.ds(ebase, 128)], src_v[0])
        pltpu.sync_copy(dst_hbm.at[pl.ds(ebase, 128)], dst_v[0])
        pltpu.async_copy(g.at[src_v[0]], rows_v[0], sems[0])
        pltpu.sync_copy(src_hbm.at[pl.ds(ebase + 128, 128)], src_v[1])
        pltpu.sync_copy(dst_hbm.at[pl.ds(ebase + 128, 128)], dst_v[1])

        def pair(s, _):
            for b in range(2):
                t = s + b
                o = (b + 1) % 2

                @pl.when(t < STEPS)
                def _():
                    pltpu.make_async_copy(g.at[src_v[b]], rows_v[b],
                                          sems[b]).wait()

                    @pl.when(t + 1 < STEPS)
                    def _():
                        pltpu.async_copy(g.at[src_v[o]], rows_v[o], sems[o])

                    pltpu.sync_copy(rows_v[b], acc.at[dst_v[b]], add=True)

                    @pl.when(t + 2 < STEPS)
                    def _():
                        off = pl.multiple_of(ebase + (t + 2) * 128, 128)
                        pltpu.sync_copy(src_hbm.at[pl.ds(off, 128)], src_v[b])
                        pltpu.sync_copy(dst_hbm.at[pl.ds(off, 128)], dst_v[b])
            return 0

        lax.fori_loop(0, (STEPS + 1) // 2, lambda i, cc: pair(2 * i, cc), 0)
        plsc.subcore_barrier()
        pltpu.sync_copy(acc.at[pl.ds(rbase, ROWS_PER_SUB)],
                        out_refs[ch].at[c, pl.ds(rbase, ROWS_PER_SUB)])
        plsc.subcore_barrier()


def _sc_scatter(nch, g_list, src, dst, zeros):
    mesh = plsc.VectorSubcoreMesh(core_axis_name="c", subcore_axis_name="s")
    kern = pl.kernel(
        functools.partial(_sc_scatter_body, nch),
        out_type=[jax.ShapeDtypeStruct((2, NPAD, 128), jnp.float32)
                  for _ in range(nch)],
        mesh=mesh,
        scratch_types=[
            pltpu.VMEM((128,), jnp.int32),
            pltpu.VMEM((128,), jnp.int32),
            pltpu.VMEM((128,), jnp.int32),
            pltpu.VMEM((128,), jnp.int32),
            pltpu.VMEM((128, 128), jnp.float32),
            pltpu.VMEM((128, 128), jnp.float32),
            pltpu.SemaphoreType.DMA,
            pltpu.SemaphoreType.DMA,
            pltpu.VMEM_SHARED((NPAD, 128), jnp.float32),
        ],
    )
    return kern(*g_list, src, dst, zeros)


def _sc_deg_body(dst_hbm, ones_hbm, zeros_hbm, out_ref,
                 dst_v, ones_v, acc):
    c = lax.axis_index("c")
    sid = lax.axis_index("s")
    ebase = pl.multiple_of(c * (16 * EPW) + sid * EPW, 128)
    rbase = pl.multiple_of(sid * ROWS_PER_SUB, 8)

    pltpu.sync_copy(ones_hbm, ones_v)
    pltpu.sync_copy(zeros_hbm.at[pl.ds(rbase, ROWS_PER_SUB)],
                    acc.at[pl.ds(rbase, ROWS_PER_SUB)])
    plsc.subcore_barrier()

    def step(t, _):
        off = pl.multiple_of(ebase + t * 128, 128)
        pltpu.sync_copy(dst_hbm.at[pl.ds(off, 128)], dst_v)
        pltpu.sync_copy(ones_v, acc.at[dst_v], add=True)
        return 0

    lax.fori_loop(0, STEPS, step, 0)
    plsc.subcore_barrier()
    pltpu.sync_copy(acc.at[pl.ds(rbase, ROWS_PER_SUB)],
                    out_ref.at[c, pl.ds(rbase, ROWS_PER_SUB)])


def _sc_deg(dst, ones, zeros):
    mesh = plsc.VectorSubcoreMesh(core_axis_name="c", subcore_axis_name="s")
    kern = pl.kernel(
        _sc_deg_body,
        out_type=jax.ShapeDtypeStruct((2, NPAD, 128), jnp.float32),
        mesh=mesh,
        scratch_types=[
            pltpu.VMEM((128,), jnp.int32),
            pltpu.VMEM((128, 128), jnp.float32),
            pltpu.VMEM_SHARED((NPAD, 128), jnp.float32),
        ],
    )
    return kern(dst, ones, zeros)


def _dinv_body(deg_ref, dinv_ref):
    d = deg_ref[0] + deg_ref[1]
    dinv_ref[...] = jnp.where(d > 0, lax.rsqrt(d), 0.0)


def _dinv(deg):
    return pl.pallas_call(
        _dinv_body,
        grid=(NRB,),
        in_specs=[pl.BlockSpec((2, ROWB, 128), lambda r: (0, r, 0))],
        out_specs=pl.BlockSpec((ROWB, 128), lambda r: (r, 0)),
        out_shape=jax.ShapeDtypeStruct((NPAD, 128), jnp.float32),
    )(deg)


def _mm_body(nch, h_ref, w_ref, dinv_ref, g_ref):
    acc = jnp.dot(h_ref[...], w_ref[...],
                  preferred_element_type=jnp.float32,
                  precision=lax.Precision.HIGHEST)
    dinv = dinv_ref[...]
    for ch in range(nch):
        g_ref[ch] = acc[:, ch * 128:(ch + 1) * 128] * dinv


def _mm_scale(h, w, dinv):
    k, dout = w.shape
    nch = dout // 128
    return pl.pallas_call(
        functools.partial(_mm_body, nch),
        grid=(NRB,),
        in_specs=[
            pl.BlockSpec((ROWB, k), lambda r: (r, 0)),
            pl.BlockSpec((k, dout), lambda r: (0, 0)),
            pl.BlockSpec((ROWB, 128), lambda r: (r, 0)),
        ],
        out_specs=pl.BlockSpec((nch, ROWB, 128), lambda r: (0, r, 0)),
        out_shape=jax.ShapeDtypeStruct((nch, NPAD, 128), jnp.float32),
    )(h, w, dinv)


def _epi_body(nch, *refs):
    p_refs = refs[:nch]
    b_ref, dinv_ref = refs[nch], refs[nch + 1]
    out_ref = refs[nch + 2]
    dinv = dinv_ref[...]
    for ch in range(nch):
        s = p_refs[ch][0] + p_refs[ch][1]
        out_ref[:, ch * 128:(ch + 1) * 128] = jax.nn.relu(
            dinv * s + b_ref[0, ch * 128:(ch + 1) * 128][None, :])


def _epilogue(p_list, b, dinv):
    nch = len(p_list)
    dout = nch * 128
    return pl.pallas_call(
        functools.partial(_epi_body, nch),
        grid=(NRB,),
        in_specs=(
            [pl.BlockSpec((2, ROWB, 128), lambda r: (0, r, 0))
             for _ in range(nch)]
            + [pl.BlockSpec((1, dout), lambda r: (0, 0)),
               pl.BlockSpec((ROWB, 128), lambda r: (r, 0))]
        ),
        out_specs=pl.BlockSpec((ROWB, dout), lambda r: (r, 0)),
        out_shape=jax.ShapeDtypeStruct((NPAD, dout), jnp.float32),
    )(*p_list, b.reshape(1, dout), dinv)


def _pool_body(h_ref, batch_ref, out_ref):
    r = pl.program_id(0)

    @pl.when(r == 0)
    def _():
        out_ref[...] = jnp.zeros_like(out_ref)

    h = h_ref[...]
    b = batch_ref[...]
    smin = jnp.min(b)
    smax = jnp.minimum(jnp.max(b), NSEG - 1)
    segcol = lax.broadcasted_iota(jnp.int32, (NSEG, 1), 0)

    def body(g, res):
        mrow = jnp.max(jnp.where(b == g, h, 0.0), axis=0, keepdims=True)
        return jnp.where(segcol == g, jnp.maximum(res, mrow), res)

    res = lax.fori_loop(smin, smax + 1, body,
                        jnp.zeros_like(out_ref[...]))
    out_ref[...] = jnp.maximum(out_ref[...], res)


def _pool(h3, batch2d):
    return pl.pallas_call(
        _pool_body,
        grid=(NRB,),
        in_specs=[pl.BlockSpec((ROWB, 512), lambda r: (r, 0)),
                  pl.BlockSpec((ROWB, 1), lambda r: (r, 0))],
        out_specs=pl.BlockSpec((NSEG, 512), lambda r: (0, 0)),
        out_shape=jax.ShapeDtypeStruct((NSEG, 512), jnp.float32),
    )(h3, batch2d)


def _mlp_body(p_ref, w1_ref, b1_ref, w2_ref, b2_ref, out_ref):
    z = jax.nn.relu(
        jnp.dot(p_ref[...], w1_ref[...],
                preferred_element_type=jnp.float32,
                precision=lax.Precision.HIGHEST) + b1_ref[...])
    out_ref[...] = jax.nn.relu(
        jnp.dot(z, w2_ref[...],
                preferred_element_type=jnp.float32,
                precision=lax.Precision.HIGHEST) + b2_ref[...])


def _mlp(pooled, wg1, bg1, wg2, bg2):
    return pl.pallas_call(
        _mlp_body,
        out_shape=jax.ShapeDtypeStruct((NSEG, 128), jnp.float32),
    )(pooled, wg1, bg1.reshape(1, 1024), wg2, bg2.reshape(1, 128))


def kernel(x, edge_index, batch, W1, b1, W2, b2, W3, b3, Wg1, bg1, Wg2, bg2):
    loop = jnp.arange(NNODE, dtype=edge_index.dtype)
    padv = jnp.full((E2P - E2,), NNODE, dtype=edge_index.dtype)
    src = jnp.concatenate([edge_index[0], loop, padv]).astype(jnp.int32)
    dst = jnp.concatenate([edge_index[1], loop, padv]).astype(jnp.int32)

    xp = jnp.zeros((NPAD, 128), x.dtype).at[:NNODE].set(x)
    zeros = jnp.zeros((NPAD, 128), jnp.float32)
    ones = jnp.ones((128, 128), jnp.float32)
    batch2d = jnp.full((NPAD, 1), NSEG, jnp.int32).at[:NNODE, 0].set(
        batch.astype(jnp.int32))

    deg = _sc_deg(dst, ones, zeros)
    dinv = _dinv(deg)

    h = xp
    for w, b in ((W1, b1), (W2, b2), (W3, b3)):
        nch = w.shape[1] // 128
        g = _mm_scale(h, w, dinv)
        g_list = [g[ch] for ch in range(nch)]
        p_list = _sc_scatter(nch, g_list, src, dst, zeros)
        if not isinstance(p_list, (list, tuple)):
            p_list = [p_list]
        p_list = list(p_list)
        h = _epilogue(p_list, b, dinv)

    pooled = _pool(h, batch2d)
    return _mlp(pooled, Wg1, bg1, Wg2, bg2)


# speedup vs baseline: 8.6015x; 1.0646x over previous
import functools

import jax
import jax.numpy as jnp
from jax import lax
from jax.experimental import pallas as pl
from jax.experimental.pallas import tpu as pltpu
from jax.experimental.pallas import tpu_sc as plsc

NNODE = 10000
NPAD = 10240
ROWB = 2048
NRB = NPAD // ROWB
NSEG = 256
EDGES = 320000
E2 = EDGES + NNODE
EPW = 10368
E2P = EPW * 32
STEPS = EPW // 128
ROWS_PER_SUB = NPAD // 16


def _sc_scatter_body(nch, *refs):
    g_refs = refs[:nch]
    src_hbm, dst_hbm, zeros_hbm = refs[nch:nch + 3]
    out_refs = refs[nch + 3:2 * nch + 3]
    src_v = refs[2 * nch + 3:2 * nch + 5]
    dst_v = refs[2 * nch + 5:2 * nch + 7]
    rows_v = refs[2 * nch + 7:2 * nch + 9]
    sems = refs[2 * nch + 9:2 * nch + 11]
    ssems = refs[2 * nch + 11:2 * nch + 13]
    acc = refs[2 * nch + 13]

    c = lax.axis_index("c")
    sid = lax.axis_index("s")
    ebase = pl.multiple_of(c * (16 * EPW) + sid * EPW, 128)
    rbase = pl.multiple_of(sid * ROWS_PER_SUB, 8)

    for ch in range(nch):
        g = g_refs[ch]
        pltpu.sync_copy(zeros_hbm.at[pl.ds(rbase, ROWS_PER_SUB)],
                        acc.at[pl.ds(rbase, ROWS_PER_SUB)])
        plsc.subcore_barrier()

        pltpu.sync_copy(src_hbm.at[pl.ds(ebase, 128)], src_v[0])
        pltpu.sync_copy(dst_hbm.at[pl.ds(ebase, 128)], dst_v[0])
        pltpu.async_copy(g.at[src_v[0]], rows_v[0], sems[0])
        pltpu.sync_copy(src_hbm.at[pl.ds(ebase + 128, 128)], src_v[1])

        def pair(s, _):
            for b in range(2):
                t = s + b
                o = (b + 1) % 2

                @pl.when(t < STEPS)
                def _():
                    pltpu.make_async_copy(g.at[src_v[b]], rows_v[b],
                                          sems[b]).wait()

                    @pl.when(t + 1 < STEPS)
                    def _():
                        @pl.when(t >= 1)
                        def _():
                            pltpu.make_async_copy(
                                rows_v[o], acc.at[dst_v[o]], ssems[o]).wait()

                        pltpu.async_copy(g.at[src_v[o]], rows_v[o], sems[o])
                        off1 = pl.multiple_of(ebase + (t + 1) * 128, 128)
                        pltpu.sync_copy(dst_hbm.at[pl.ds(off1, 128)],
                                        dst_v[o])

                    pltpu.async_copy(rows_v[b], acc.at[dst_v[b]], ssems[b],
                                     add=True)

                    @pl.when(t + 2 < STEPS)
                    def _():
                        off = pl.multiple_of(ebase + (t + 2) * 128, 128)
                        pltpu.sync_copy(src_hbm.at[pl.ds(off, 128)], src_v[b])
            return 0

        lax.fori_loop(0, (STEPS + 1) // 2, lambda i, cc: pair(2 * i, cc), 0)
        pltpu.make_async_copy(rows_v[0], acc.at[dst_v[0]], ssems[0]).wait()
        pltpu.make_async_copy(rows_v[1], acc.at[dst_v[1]], ssems[1]).wait()
        plsc.subcore_barrier()
        pltpu.sync_copy(acc.at[pl.ds(rbase, ROWS_PER_SUB)],
                        out_refs[ch].at[c, pl.ds(rbase, ROWS_PER_SUB)])
        plsc.subcore_barrier()


def _sc_scatter(nch, g_list, src, dst, zeros):
    mesh = plsc.VectorSubcoreMesh(core_axis_name="c", subcore_axis_name="s")
    kern = pl.kernel(
        functools.partial(_sc_scatter_body, nch),
        out_type=[jax.ShapeDtypeStruct((2, NPAD, 128), jnp.float32)
                  for _ in range(nch)],
        mesh=mesh,
        scratch_types=[
            pltpu.VMEM((128,), jnp.int32),
            pltpu.VMEM((128,), jnp.int32),
            pltpu.VMEM((128,), jnp.int32),
            pltpu.VMEM((128,), jnp.int32),
            pltpu.VMEM((128, 128), jnp.float32),
            pltpu.VMEM((128, 128), jnp.float32),
            pltpu.SemaphoreType.DMA,
            pltpu.SemaphoreType.DMA,
            pltpu.SemaphoreType.DMA,
            pltpu.SemaphoreType.DMA,
            pltpu.VMEM_SHARED((NPAD, 128), jnp.float32),
        ],
    )
    return kern(*g_list, src, dst, zeros)


def _sc_deg_body(dst_hbm, ones_hbm, zeros_hbm, out_ref,
                 dst_v, ones_v, acc):
    c = lax.axis_index("c")
    sid = lax.axis_index("s")
    ebase = pl.multiple_of(c * (16 * EPW) + sid * EPW, 128)
    rbase = pl.multiple_of(sid * ROWS_PER_SUB, 8)

    pltpu.sync_copy(ones_hbm, ones_v)
    pltpu.sync_copy(zeros_hbm.at[pl.ds(rbase, ROWS_PER_SUB)],
                    acc.at[pl.ds(rbase, ROWS_PER_SUB)])
    plsc.subcore_barrier()

    def step(t, _):
        off = pl.multiple_of(ebase + t * 128, 128)
        pltpu.sync_copy(dst_hbm.at[pl.ds(off, 128)], dst_v)
        pltpu.sync_copy(ones_v, acc.at[dst_v], add=True)
        return 0

    lax.fori_loop(0, STEPS, step, 0)
    plsc.subcore_barrier()
    pltpu.sync_copy(acc.at[pl.ds(rbase, ROWS_PER_SUB)],
                    out_ref.at[c, pl.ds(rbase, ROWS_PER_SUB)])


def _sc_deg(dst, ones, zeros):
    mesh = plsc.VectorSubcoreMesh(core_axis_name="c", subcore_axis_name="s")
    kern = pl.kernel(
        _sc_deg_body,
        out_type=jax.ShapeDtypeStruct((2, NPAD, 128), jnp.float32),
        mesh=mesh,
        scratch_types=[
            pltpu.VMEM((128,), jnp.int32),
            pltpu.VMEM((128, 128), jnp.float32),
            pltpu.VMEM_SHARED((NPAD, 128), jnp.float32),
        ],
    )
    return kern(dst, ones, zeros)


def _dinv_body(deg_ref, dinv_ref):
    d = deg_ref[0] + deg_ref[1]
    dinv_ref[...] = jnp.where(d > 0, lax.rsqrt(d), 0.0)


def _dinv(deg):
    return pl.pallas_call(
        _dinv_body,
        grid=(NRB,),
        in_specs=[pl.BlockSpec((2, ROWB, 128), lambda r: (0, r, 0))],
        out_specs=pl.BlockSpec((ROWB, 128), lambda r: (r, 0)),
        out_shape=jax.ShapeDtypeStruct((NPAD, 128), jnp.float32),
    )(deg)


def _mm_body(nch, h_ref, w_ref, dinv_ref, g_ref):
    acc = jnp.dot(h_ref[...], w_ref[...],
                  preferred_element_type=jnp.float32,
                  precision=lax.Precision.HIGHEST)
    dinv = dinv_ref[...]
    for ch in range(nch):
        g_ref[ch] = acc[:, ch * 128:(ch + 1) * 128] * dinv


def _mm_scale(h, w, dinv):
    k, dout = w.shape
    nch = dout // 128
    return pl.pallas_call(
        functools.partial(_mm_body, nch),
        grid=(NRB,),
        in_specs=[
            pl.BlockSpec((ROWB, k), lambda r: (r, 0)),
            pl.BlockSpec((k, dout), lambda r: (0, 0)),
            pl.BlockSpec((ROWB, 128), lambda r: (r, 0)),
        ],
        out_specs=pl.BlockSpec((nch, ROWB, 128), lambda r: (0, r, 0)),
        out_shape=jax.ShapeDtypeStruct((nch, NPAD, 128), jnp.float32),
    )(h, w, dinv)


def _epi_body(nch, *refs):
    p_refs = refs[:nch]
    b_ref, dinv_ref = refs[nch], refs[nch + 1]
    out_ref = refs[nch + 2]
    dinv = dinv_ref[...]
    for ch in range(nch):
        s = p_refs[ch][0] + p_refs[ch][1]
        out_ref[:, ch * 128:(ch + 1) * 128] = jax.nn.relu(
            dinv * s + b_ref[0, ch * 128:(ch + 1) * 128][None, :])


def _epilogue(p_list, b, dinv):
    nch = len(p_list)
    dout = nch * 128
    return pl.pallas_call(
        functools.partial(_epi_body, nch),
        grid=(NRB,),
        in_specs=(
            [pl.BlockSpec((2, ROWB, 128), lambda r: (0, r, 0))
             for _ in range(nch)]
            + [pl.BlockSpec((1, dout), lambda r: (0, 0)),
               pl.BlockSpec((ROWB, 128), lambda r: (r, 0))]
        ),
        out_specs=pl.BlockSpec((ROWB, dout), lambda r: (r, 0)),
        out_shape=jax.ShapeDtypeStruct((NPAD, dout), jnp.float32),
    )(*p_list, b.reshape(1, dout), dinv)


def _pool_body(h_ref, batch_ref, out_ref):
    r = pl.program_id(0)

    @pl.when(r == 0)
    def _():
        out_ref[...] = jnp.zeros_like(out_ref)

    h = h_ref[...]
    b = batch_ref[...]
    smin = jnp.min(b)
    smax = jnp.minimum(jnp.max(b), NSEG - 1)
    segcol = lax.broadcasted_iota(jnp.int32, (NSEG, 1), 0)

    def body(g, res):
        mrow = jnp.max(jnp.where(b == g, h, 0.0), axis=0, keepdims=True)
        return jnp.where(segcol == g, jnp.maximum(res, mrow), res)

    res = lax.fori_loop(smin, smax + 1, body,
                        jnp.zeros_like(out_ref[...]))
    out_ref[...] = jnp.maximum(out_ref[...], res)


def _pool(h3, batch2d):
    return pl.pallas_call(
        _pool_body,
        grid=(NRB,),
        in_specs=[pl.BlockSpec((ROWB, 512), lambda r: (r, 0)),
                  pl.BlockSpec((ROWB, 1), lambda r: (r, 0))],
        out_specs=pl.BlockSpec((NSEG, 512), lambda r: (0, 0)),
        out_shape=jax.ShapeDtypeStruct((NSEG, 512), jnp.float32),
    )(h3, batch2d)


def _mlp_body(p_ref, w1_ref, b1_ref, w2_ref, b2_ref, out_ref):
    z = jax.nn.relu(
        jnp.dot(p_ref[...], w1_ref[...],
                preferred_element_type=jnp.float32,
                precision=lax.Precision.HIGHEST) + b1_ref[...])
    out_ref[...] = jax.nn.relu(
        jnp.dot(z, w2_ref[...],
                preferred_element_type=jnp.float32,
                precision=lax.Precision.HIGHEST) + b2_ref[...])


def _mlp(pooled, wg1, bg1, wg2, bg2):
    return pl.pallas_call(
        _mlp_body,
        out_shape=jax.ShapeDtypeStruct((NSEG, 128), jnp.float32),
    )(pooled, wg1, bg1.reshape(1, 1024), wg2, bg2.reshape(1, 128))


def kernel(x, edge_index, batch, W1, b1, W2, b2, W3, b3, Wg1, bg1, Wg2, bg2):
    loop = jnp.arange(NNODE, dtype=edge_index.dtype)
    padv = jnp.full((E2P - E2,), NNODE, dtype=edge_index.dtype)
    src = jnp.concatenate([edge_index[0], loop, padv]).astype(jnp.int32)
    dst = jnp.concatenate([edge_index[1], loop, padv]).astype(jnp.int32)

    xp = jnp.zeros((NPAD, 128), x.dtype).at[:NNODE].set(x)
    zeros = jnp.zeros((NPAD, 128), jnp.float32)
    ones = jnp.ones((128, 128), jnp.float32)
    batch2d = jnp.full((NPAD, 1), NSEG, jnp.int32).at[:NNODE, 0].set(
        batch.astype(jnp.int32))

    deg = _sc_deg(dst, ones, zeros)
    dinv = _dinv(deg)

    h = xp
    for w, b in ((W1, b1), (W2, b2), (W3, b3)):
        nch = w.shape[1] // 128
        g = _mm_scale(h, w, dinv)
        g_list = [g[ch] for ch in range(nch)]
        p_list = _sc_scatter(nch, g_list, src, dst, zeros)
        if not isinstance(p_list, (list, tuple)):
            p_list = [p_list]
        p_list = list(p_list)
        h = _epilogue(p_list, b, dinv)

    pooled = _pool(h, batch2d)
    return _mlp(pooled, Wg1, bg1, Wg2, bg2)


# speedup vs baseline: 9.3559x; 1.0877x over previous
import functools

import jax
import jax.numpy as jnp
from jax import lax
from jax.experimental import pallas as pl
from jax.experimental.pallas import tpu as pltpu
from jax.experimental.pallas import tpu_sc as plsc

NNODE = 10000
NPAD = 10240
ROWB = 2048
NRB = NPAD // ROWB
NSEG = 256
EDGES = 320000
E2 = EDGES + NNODE
EPW = 10368
E2P = EPW * 32
STEPS = EPW // 128
ROWS_PER_SUB = NPAD // 16


def _sc_scatter_body(nch, *refs):
    g_refs = refs[:nch]
    src_hbm, dst_hbm, zeros_hbm = refs[nch:nch + 3]
    out_refs = refs[nch + 3:2 * nch + 3]
    src_v = refs[2 * nch + 3:2 * nch + 5]
    dst_v = refs[2 * nch + 5:2 * nch + 7]
    rows_v = refs[2 * nch + 7:2 * nch + 9]
    sems = refs[2 * nch + 9:2 * nch + 11]
    ssems = refs[2 * nch + 11:2 * nch + 13]
    acc = refs[2 * nch + 13]

    c = lax.axis_index("c")
    sid = lax.axis_index("s")
    wbase = pl.multiple_of((c * 16 + sid) * 128, 128)
    rbase = pl.multiple_of(sid * ROWS_PER_SUB, 8)

    for ch in range(nch):
        g = g_refs[ch]
        pltpu.sync_copy(zeros_hbm.at[pl.ds(rbase, ROWS_PER_SUB)],
                        acc.at[pl.ds(rbase, ROWS_PER_SUB)])
        plsc.subcore_barrier()

        pltpu.sync_copy(src_hbm.at[pl.ds(wbase, 128)], src_v[0])
        pltpu.sync_copy(dst_hbm.at[pl.ds(wbase, 128)], dst_v[0])
        pltpu.async_copy(g.at[src_v[0]], rows_v[0], sems[0])
        pltpu.sync_copy(src_hbm.at[pl.ds(wbase + 4096, 128)], src_v[1])

        def pair(s, _):
            for b in range(2):
                t = s + b
                o = (b + 1) % 2

                @pl.when(t < STEPS)
                def _():
                    pltpu.make_async_copy(g.at[src_v[b]], rows_v[b],
                                          sems[b]).wait()

                    @pl.when(t + 1 < STEPS)
                    def _():
                        @pl.when(t >= 1)
                        def _():
                            pltpu.make_async_copy(
                                rows_v[o], acc.at[dst_v[o]], ssems[o]).wait()

                        pltpu.async_copy(g.at[src_v[o]], rows_v[o], sems[o])
                        off1 = pl.multiple_of(wbase + (t + 1) * 4096, 128)
                        pltpu.sync_copy(dst_hbm.at[pl.ds(off1, 128)],
                                        dst_v[o])

                    pltpu.async_copy(rows_v[b], acc.at[dst_v[b]], ssems[b],
                                     add=True)

                    @pl.when(t + 2 < STEPS)
                    def _():
                        off = pl.multiple_of(wbase + (t + 2) * 4096, 128)
                        pltpu.sync_copy(src_hbm.at[pl.ds(off, 128)], src_v[b])
            return 0

        lax.fori_loop(0, (STEPS + 1) // 2, lambda i, cc: pair(2 * i, cc), 0)
        pltpu.make_async_copy(rows_v[0], acc.at[dst_v[0]], ssems[0]).wait()
        pltpu.make_async_copy(rows_v[1], acc.at[dst_v[1]], ssems[1]).wait()
        plsc.subcore_barrier()
        pltpu.sync_copy(acc.at[pl.ds(rbase, ROWS_PER_SUB)],
                        out_refs[ch].at[c, pl.ds(rbase, ROWS_PER_SUB)])
        plsc.subcore_barrier()


def _sc_scatter(nch, g_list, src, dst, zeros):
    mesh = plsc.VectorSubcoreMesh(core_axis_name="c", subcore_axis_name="s")
    kern = pl.kernel(
        functools.partial(_sc_scatter_body, nch),
        out_type=[jax.ShapeDtypeStruct((2, NPAD, 128), jnp.float32)
                  for _ in range(nch)],
        mesh=mesh,
        scratch_types=[
            pltpu.VMEM((128,), jnp.int32),
            pltpu.VMEM((128,), jnp.int32),
            pltpu.VMEM((128,), jnp.int32),
            pltpu.VMEM((128,), jnp.int32),
            pltpu.VMEM((128, 128), jnp.float32),
            pltpu.VMEM((128, 128), jnp.float32),
            pltpu.SemaphoreType.DMA,
            pltpu.SemaphoreType.DMA,
            pltpu.SemaphoreType.DMA,
            pltpu.SemaphoreType.DMA,
            pltpu.VMEM_SHARED((NPAD, 128), jnp.float32),
        ],
    )
    return kern(*g_list, src, dst, zeros)


def _sc_deg_body(dst_hbm, ones_hbm, zeros_hbm, out_ref,
                 dst_v, ones_v, acc):
    c = lax.axis_index("c")
    sid = lax.axis_index("s")
    wbase = pl.multiple_of((c * 16 + sid) * 128, 128)
    rbase = pl.multiple_of(sid * ROWS_PER_SUB, 8)

    pltpu.sync_copy(ones_hbm, ones_v)
    pltpu.sync_copy(zeros_hbm.at[pl.ds(rbase, ROWS_PER_SUB)],
                    acc.at[pl.ds(rbase, ROWS_PER_SUB)])
    plsc.subcore_barrier()

    def step(t, _):
        off = pl.multiple_of(wbase + t * 4096, 128)
        pltpu.sync_copy(dst_hbm.at[pl.ds(off, 128)], dst_v)
        pltpu.sync_copy(ones_v, acc.at[dst_v], add=True)
        return 0

    lax.fori_loop(0, STEPS, step, 0)
    plsc.subcore_barrier()
    pltpu.sync_copy(acc.at[pl.ds(rbase, ROWS_PER_SUB)],
                    out_ref.at[c, pl.ds(rbase, ROWS_PER_SUB)])


def _sc_deg(dst, ones, zeros):
    mesh = plsc.VectorSubcoreMesh(core_axis_name="c", subcore_axis_name="s")
    kern = pl.kernel(
        _sc_deg_body,
        out_type=jax.ShapeDtypeStruct((2, NPAD, 128), jnp.float32),
        mesh=mesh,
        scratch_types=[
            pltpu.VMEM((128,), jnp.int32),
            pltpu.VMEM((128, 128), jnp.float32),
            pltpu.VMEM_SHARED((NPAD, 128), jnp.float32),
        ],
    )
    return kern(dst, ones, zeros)


def _dinv_body(deg_ref, dinv_ref):
    d = deg_ref[0] + deg_ref[1]
    dinv_ref[...] = jnp.where(d > 0, lax.rsqrt(d), 0.0)


def _dinv(deg):
    return pl.pallas_call(
        _dinv_body,
        grid=(NRB,),
        in_specs=[pl.BlockSpec((2, ROWB, 128), lambda r: (0, r, 0))],
        out_specs=pl.BlockSpec((ROWB, 128), lambda r: (r, 0)),
        out_shape=jax.ShapeDtypeStruct((NPAD, 128), jnp.float32),
    )(deg)


def _mm_body(nch, h_ref, w_ref, dinv_ref, g_ref):
    acc = jnp.dot(h_ref[...], w_ref[...],
                  preferred_element_type=jnp.float32,
                  precision=lax.Precision.HIGHEST)
    dinv = dinv_ref[...]
    for ch in range(nch):
        g_ref[ch] = acc[:, ch * 128:(ch + 1) * 128] * dinv


def _mm_scale(h, w, dinv):
    k, dout = w.shape
    nch = dout // 128
    return pl.pallas_call(
        functools.partial(_mm_body, nch),
        grid=(NRB,),
        in_specs=[
            pl.BlockSpec((ROWB, k), lambda r: (r, 0)),
            pl.BlockSpec((k, dout), lambda r: (0, 0)),
            pl.BlockSpec((ROWB, 128), lambda r: (r, 0)),
        ],
        out_specs=pl.BlockSpec((nch, ROWB, 128), lambda r: (0, r, 0)),
        out_shape=jax.ShapeDtypeStruct((nch, NPAD, 128), jnp.float32),
    )(h, w, dinv)


def _epi_body(nch, *refs):
    p_refs = refs[:nch]
    b_ref, dinv_ref = refs[nch], refs[nch + 1]
    out_ref = refs[nch + 2]
    dinv = dinv_ref[...]
    for ch in range(nch):
        s = p_refs[ch][0] + p_refs[ch][1]
        out_ref[:, ch * 128:(ch + 1) * 128] = jax.nn.relu(
            dinv * s + b_ref[0, ch * 128:(ch + 1) * 128][None, :])


def _epilogue(p_list, b, dinv):
    nch = len(p_list)
    dout = nch * 128
    return pl.pallas_call(
        functools.partial(_epi_body, nch),
        grid=(NRB,),
        in_specs=(
            [pl.BlockSpec((2, ROWB, 128), lambda r: (0, r, 0))
             for _ in range(nch)]
            + [pl.BlockSpec((1, dout), lambda r: (0, 0)),
               pl.BlockSpec((ROWB, 128), lambda r: (r, 0))]
        ),
        out_specs=pl.BlockSpec((ROWB, dout), lambda r: (r, 0)),
        out_shape=jax.ShapeDtypeStruct((NPAD, dout), jnp.float32),
    )(*p_list, b.reshape(1, dout), dinv)


def _pool_body(h_ref, batch_ref, out_ref):
    r = pl.program_id(0)

    @pl.when(r == 0)
    def _():
        out_ref[...] = jnp.zeros_like(out_ref)

    h = h_ref[...]
    b = batch_ref[...]
    smin = jnp.min(b)
    smax = jnp.minimum(jnp.max(b), NSEG - 1)
    segcol = lax.broadcasted_iota(jnp.int32, (NSEG, 1), 0)

    def body(g, res):
        mrow = jnp.max(jnp.where(b == g, h, 0.0), axis=0, keepdims=True)
        return jnp.where(segcol == g, jnp.maximum(res, mrow), res)

    res = lax.fori_loop(smin, smax + 1, body,
                        jnp.zeros_like(out_ref[...]))
    out_ref[...] = jnp.maximum(out_ref[...], res)


def _pool(h3, batch2d):
    return pl.pallas_call(
        _pool_body,
        grid=(NRB,),
        in_specs=[pl.BlockSpec((ROWB, 512), lambda r: (r, 0)),
                  pl.BlockSpec((ROWB, 1), lambda r: (r, 0))],
        out_specs=pl.BlockSpec((NSEG, 512), lambda r: (0, 0)),
        out_shape=jax.ShapeDtypeStruct((NSEG, 512), jnp.float32),
    )(h3, batch2d)


def _mlp_body(p_ref, w1_ref, b1_ref, w2_ref, b2_ref, out_ref):
    z = jax.nn.relu(
        jnp.dot(p_ref[...], w1_ref[...],
                preferred_element_type=jnp.float32,
                precision=lax.Precision.HIGHEST) + b1_ref[...])
    out_ref[...] = jax.nn.relu(
        jnp.dot(z, w2_ref[...],
                preferred_element_type=jnp.float32,
                precision=lax.Precision.HIGHEST) + b2_ref[...])


def _mlp(pooled, wg1, bg1, wg2, bg2):
    return pl.pallas_call(
        _mlp_body,
        out_shape=jax.ShapeDtypeStruct((NSEG, 128), jnp.float32),
    )(pooled, wg1, bg1.reshape(1, 1024), wg2, bg2.reshape(1, 128))


def kernel(x, edge_index, batch, W1, b1, W2, b2, W3, b3, Wg1, bg1, Wg2, bg2):
    loop = jnp.arange(NNODE, dtype=edge_index.dtype)
    padv = jnp.full((E2P - E2,), NNODE, dtype=edge_index.dtype)
    src = jnp.concatenate([edge_index[0], loop, padv]).astype(jnp.int32)
    dst = jnp.concatenate([edge_index[1], loop, padv]).astype(jnp.int32)

    xp = jnp.zeros((NPAD, 128), x.dtype).at[:NNODE].set(x)
    zeros = jnp.zeros((NPAD, 128), jnp.float32)
    ones = jnp.ones((128, 128), jnp.float32)
    batch2d = jnp.full((NPAD, 1), NSEG, jnp.int32).at[:NNODE, 0].set(
        batch.astype(jnp.int32))

    deg = _sc_deg(dst, ones, zeros)
    dinv = _dinv(deg)

    h = xp
    for w, b in ((W1, b1), (W2, b2), (W3, b3)):
        nch = w.shape[1] // 128
        g = _mm_scale(h, w, dinv)
        g_list = [g[ch] for ch in range(nch)]
        p_list = _sc_scatter(nch, g_list, src, dst, zeros)
        if not isinstance(p_list, (list, tuple)):
            p_list = [p_list]
        p_list = list(p_list)
        h = _epilogue(p_list, b, dinv)

    pooled = _pool(h, batch2d)
    return _mlp(pooled, Wg1, bg1, Wg2, bg2)
